# baseline (speedup 1.0000x reference)
import functools

import jax
import jax.numpy as jnp
from jax.experimental import pallas as pl
from jax.experimental.pallas import tpu as pltpu

B = 64
T = 200
AD = 128
ED = 128
OD = 2048

_DN = (((1,), (1,)), ((), ()))


def _fused_kernel(a_feat_ref, v_feat_ref, mask_a_ref, mask_v_ref,
                  modal_ref, time_ref, pad_ref,
                  Wa_f_ref, Wa_e_ref, ba_ref, Wv_f_ref, Wv_e_ref, bv_ref,
                  out_ref,
                  base_a_ref, base_v_ref, wpad_a_ref, wpad_v_ref):
    b = pl.program_id(0)

    @pl.when(b == 0)
    def _():
        te = time_ref[...]
        ea = te + modal_ref[0:1, :]
        ev = te + modal_ref[1:2, :]
        Wae = Wa_e_ref[...]
        Wve = Wv_e_ref[...]
        base_a_ref[...] = (
            jax.lax.dot_general(Wae, ea, _DN, preferred_element_type=jnp.float32)
            + ba_ref[...])
        base_v_ref[...] = (
            jax.lax.dot_general(Wve, ev, _DN, preferred_element_type=jnp.float32)
            + bv_ref[...])
        wpad_a_ref[...] = jax.lax.dot_general(
            Wae, pad_ref[...], _DN, preferred_element_type=jnp.float32)
        wpad_v_ref[...] = jax.lax.dot_general(
            Wve, pad_ref[...], _DN, preferred_element_type=jnp.float32)

    a_res = (
        jax.lax.dot_general(Wa_f_ref[...], a_feat_ref[0], _DN,
                            preferred_element_type=jnp.float32)
        + base_a_ref[...]
        + wpad_a_ref[...] * mask_a_ref[0])
    v_res = (
        jax.lax.dot_general(Wv_f_ref[...], v_feat_ref[0], _DN,
                            preferred_element_type=jnp.float32)
        + base_v_ref[...]
        + wpad_v_ref[...] * mask_v_ref[0])
    out_ref[0, :, :T] = a_res
    out_ref[0, :, T:] = v_res


@jax.jit
def kernel(audio_feat, video_feat, mask_audio, mask_video, modal_emb,
           time_emb, pad_emb, W_audio, b_audio, W_video, b_video):
    a_feat = audio_feat.astype(jnp.bfloat16)
    v_feat = video_feat.astype(jnp.bfloat16)
    mask_a = mask_audio.astype(jnp.float32).reshape(B, 1, T)
    mask_v = mask_video.astype(jnp.float32).reshape(B, 1, T)
    Wa_f = W_audio[:, :AD].astype(jnp.bfloat16)
    Wv_f = W_video[:, :AD].astype(jnp.bfloat16)
    Wa_e = W_audio[:, AD:]
    Wv_e = W_video[:, AD:]
    ba = b_audio.reshape(OD, 1)
    bv = b_video.reshape(OD, 1)

    out = pl.pallas_call(
        _fused_kernel,
        grid=(B,),
        in_specs=[
            pl.BlockSpec((1, T, AD), lambda b: (b, 0, 0)),
            pl.BlockSpec((1, T, AD), lambda b: (b, 0, 0)),
            pl.BlockSpec((1, 1, T), lambda b: (b, 0, 0)),
            pl.BlockSpec((1, 1, T), lambda b: (b, 0, 0)),
            pl.BlockSpec((2, ED), lambda b: (0, 0)),
            pl.BlockSpec((T, ED), lambda b: (0, 0)),
            pl.BlockSpec((1, ED), lambda b: (0, 0)),
            pl.BlockSpec((OD, AD), lambda b: (0, 0)),
            pl.BlockSpec((OD, ED), lambda b: (0, 0)),
            pl.BlockSpec((OD, 1), lambda b: (0, 0)),
            pl.BlockSpec((OD, AD), lambda b: (0, 0)),
            pl.BlockSpec((OD, ED), lambda b: (0, 0)),
            pl.BlockSpec((OD, 1), lambda b: (0, 0)),
        ],
        out_specs=pl.BlockSpec((1, OD, 2 * T), lambda b: (b, 0, 0)),
        out_shape=jax.ShapeDtypeStruct((B, OD, 2 * T), jnp.float32),
        scratch_shapes=[
            pltpu.VMEM((OD, T), jnp.float32),
            pltpu.VMEM((OD, T), jnp.float32),
            pltpu.VMEM((OD, 1), jnp.float32),
            pltpu.VMEM((OD, 1), jnp.float32),
        ],
    )(a_feat, v_feat, mask_a, mask_v, modal_emb, time_emb, pad_emb,
      Wa_f, Wa_e, ba, Wv_f, Wv_e, bv)
    return out


# speedup vs baseline: 3.0147x; 3.0147x over previous
import functools

import jax
import jax.numpy as jnp
from jax.experimental import pallas as pl
from jax.experimental.pallas import tpu as pltpu

B = 64
T = 200
AD = 128
ED = 128
OD = 2048

_DN = (((1,), (0,)), ((), ()))


def _fused_kernel(a_feat_ref, v_feat_ref, mask_a_ref, mask_v_ref,
                  modal_ref, time_ref, pad_ref,
                  Wa_f_ref, Wa_e_ref, ba_ref, Wv_f_ref, Wv_e_ref, bv_ref,
                  out_ref,
                  base_a_ref, base_v_ref, wpad_a_ref, wpad_v_ref):
    b = pl.program_id(0)

    @pl.when(b == 0)
    def _():
        te = time_ref[...]
        ea = te + modal_ref[0:1, :]
        ev = te + modal_ref[1:2, :]
        Wae = Wa_e_ref[...]
        Wve = Wv_e_ref[...]
        base_a_ref[...] = (
            jax.lax.dot_general(ea, Wae, _DN, preferred_element_type=jnp.float32)
            + ba_ref[...])
        base_v_ref[...] = (
            jax.lax.dot_general(ev, Wve, _DN, preferred_element_type=jnp.float32)
            + bv_ref[...])
        wpad_a_ref[...] = jax.lax.dot_general(
            pad_ref[...], Wae, _DN, preferred_element_type=jnp.float32)
        wpad_v_ref[...] = jax.lax.dot_general(
            pad_ref[...], Wve, _DN, preferred_element_type=jnp.float32)

    a_res = (
        jax.lax.dot_general(a_feat_ref[0], Wa_f_ref[...], _DN,
                            preferred_element_type=jnp.float32)
        + base_a_ref[...]
        + mask_a_ref[0] * wpad_a_ref[...])
    v_res = (
        jax.lax.dot_general(v_feat_ref[0], Wv_f_ref[...], _DN,
                            preferred_element_type=jnp.float32)
        + base_v_ref[...]
        + mask_v_ref[0] * wpad_v_ref[...])
    out_ref[0, :T, :] = a_res
    out_ref[0, T:, :] = v_res


@jax.jit
def kernel(audio_feat, video_feat, mask_audio, mask_video, modal_emb,
           time_emb, pad_emb, W_audio, b_audio, W_video, b_video):
    a_feat = audio_feat.astype(jnp.bfloat16)
    v_feat = video_feat.astype(jnp.bfloat16)
    mask_a = mask_audio.astype(jnp.float32).reshape(B, T, 1)
    mask_v = mask_video.astype(jnp.float32).reshape(B, T, 1)
    Wa_f = W_audio[:, :AD].T.astype(jnp.bfloat16)
    Wv_f = W_video[:, :AD].T.astype(jnp.bfloat16)
    Wa_e = W_audio[:, AD:].T
    Wv_e = W_video[:, AD:].T
    ba = b_audio.reshape(1, OD)
    bv = b_video.reshape(1, OD)

    out_tr = pl.pallas_call(
        _fused_kernel,
        grid=(B,),
        in_specs=[
            pl.BlockSpec((1, T, AD), lambda b: (b, 0, 0)),
            pl.BlockSpec((1, T, AD), lambda b: (b, 0, 0)),
            pl.BlockSpec((1, T, 1), lambda b: (b, 0, 0)),
            pl.BlockSpec((1, T, 1), lambda b: (b, 0, 0)),
            pl.BlockSpec((2, ED), lambda b: (0, 0)),
            pl.BlockSpec((T, ED), lambda b: (0, 0)),
            pl.BlockSpec((1, ED), lambda b: (0, 0)),
            pl.BlockSpec((AD, OD), lambda b: (0, 0)),
            pl.BlockSpec((ED, OD), lambda b: (0, 0)),
            pl.BlockSpec((1, OD), lambda b: (0, 0)),
            pl.BlockSpec((AD, OD), lambda b: (0, 0)),
            pl.BlockSpec((ED, OD), lambda b: (0, 0)),
            pl.BlockSpec((1, OD), lambda b: (0, 0)),
        ],
        out_specs=pl.BlockSpec((1, 2 * T, OD), lambda b: (b, 0, 0)),
        out_shape=jax.ShapeDtypeStruct((B, 2 * T, OD), jnp.float32),
        scratch_shapes=[
            pltpu.VMEM((T, OD), jnp.float32),
            pltpu.VMEM((T, OD), jnp.float32),
            pltpu.VMEM((1, OD), jnp.float32),
            pltpu.VMEM((1, OD), jnp.float32),
        ],
    )(a_feat, v_feat, mask_a, mask_v, modal_emb, time_emb, pad_emb,
      Wa_f, Wa_e, ba, Wv_f, Wv_e, bv)
    return jnp.transpose(out_tr, (0, 2, 1))


# speedup vs baseline: 3.7735x; 1.2517x over previous
import functools

import jax
import jax.numpy as jnp
from jax.experimental import pallas as pl
from jax.experimental.pallas import tpu as pltpu

B = 64
T = 200
AD = 128
ED = 128
OD = 2048

_DN = (((1,), (0,)), ((), ()))
_DNT = (((1,), (1,)), ((), ()))


def _fused_kernel(a_feat_ref, v_feat_ref, mask_a_ref, mask_v_ref,
                  modal_ref, time_ref, pad_ref,
                  Wa_ref, ba_ref, Wv_ref, bv_ref,
                  out_ref,
                  base_a_ref, base_v_ref, wpad_a_ref, wpad_v_ref,
                  Wa_bf_ref, Wv_bf_ref):
    b = pl.program_id(0)

    @pl.when(b == 0)
    def _():
        te = time_ref[...]
        ea = te + modal_ref[0:1, :]
        ev = te + modal_ref[1:2, :]
        Wae = Wa_ref[:, AD:]
        Wve = Wv_ref[:, AD:]
        base_a_ref[...] = (
            jax.lax.dot_general(ea, Wae, _DNT, preferred_element_type=jnp.float32)
            + ba_ref[...])
        base_v_ref[...] = (
            jax.lax.dot_general(ev, Wve, _DNT, preferred_element_type=jnp.float32)
            + bv_ref[...])
        wpad_a_ref[...] = jax.lax.dot_general(
            pad_ref[...], Wae, _DNT, preferred_element_type=jnp.float32)
        wpad_v_ref[...] = jax.lax.dot_general(
            pad_ref[...], Wve, _DNT, preferred_element_type=jnp.float32)
        Wa_bf_ref[...] = jnp.transpose(Wa_ref[:, :AD], (1, 0)).astype(jnp.bfloat16)
        Wv_bf_ref[...] = jnp.transpose(Wv_ref[:, :AD], (1, 0)).astype(jnp.bfloat16)

    mask_col_a = jnp.transpose(mask_a_ref[0], (1, 0))
    mask_col_v = jnp.transpose(mask_v_ref[0], (1, 0))
    a_res = (
        jax.lax.dot_general(a_feat_ref[0].astype(jnp.bfloat16), Wa_bf_ref[...],
                            _DN, preferred_element_type=jnp.float32)
        + base_a_ref[...]
        + mask_col_a * wpad_a_ref[...])
    v_res = (
        jax.lax.dot_general(v_feat_ref[0].astype(jnp.bfloat16), Wv_bf_ref[...],
                            _DN, preferred_element_type=jnp.float32)
        + base_v_ref[...]
        + mask_col_v * wpad_v_ref[...])
    out_ref[0, :T, :] = a_res
    out_ref[0, T:, :] = v_res


@jax.jit
def kernel(audio_feat, video_feat, mask_audio, mask_video, modal_emb,
           time_emb, pad_emb, W_audio, b_audio, W_video, b_video):
    mask_a = mask_audio.astype(jnp.float32).reshape(B, 1, T)
    mask_v = mask_video.astype(jnp.float32).reshape(B, 1, T)
    ba = b_audio.reshape(1, OD)
    bv = b_video.reshape(1, OD)

    out_tr = pl.pallas_call(
        _fused_kernel,
        grid=(B,),
        in_specs=[
            pl.BlockSpec((1, T, AD), lambda b: (b, 0, 0)),
            pl.BlockSpec((1, T, AD), lambda b: (b, 0, 0)),
            pl.BlockSpec((1, 1, T), lambda b: (b, 0, 0)),
            pl.BlockSpec((1, 1, T), lambda b: (b, 0, 0)),
            pl.BlockSpec((2, ED), lambda b: (0, 0)),
            pl.BlockSpec((T, ED), lambda b: (0, 0)),
            pl.BlockSpec((1, ED), lambda b: (0, 0)),
            pl.BlockSpec((OD, AD + ED), lambda b: (0, 0)),
            pl.BlockSpec((1, OD), lambda b: (0, 0)),
            pl.BlockSpec((OD, AD + ED), lambda b: (0, 0)),
            pl.BlockSpec((1, OD), lambda b: (0, 0)),
        ],
        out_specs=pl.BlockSpec((1, 2 * T, OD), lambda b: (b, 0, 0)),
        out_shape=jax.ShapeDtypeStruct((B, 2 * T, OD), jnp.float32),
        scratch_shapes=[
            pltpu.VMEM((T, OD), jnp.float32),
            pltpu.VMEM((T, OD), jnp.float32),
            pltpu.VMEM((1, OD), jnp.float32),
            pltpu.VMEM((1, OD), jnp.float32),
            pltpu.VMEM((AD, OD), jnp.bfloat16),
            pltpu.VMEM((AD, OD), jnp.bfloat16),
        ],
    )(audio_feat, video_feat, mask_a, mask_v, modal_emb, time_emb, pad_emb,
      W_audio, ba, W_video, bv)
    return jnp.transpose(out_tr, (0, 2, 1))


# speedup vs baseline: 4.6195x; 1.2242x over previous
import functools

import jax
import jax.numpy as jnp
from jax.experimental import pallas as pl
from jax.experimental.pallas import tpu as pltpu

B = 64
T = 200
AD = 128
ED = 128
OD = 2048

BB = 2

_DN = (((1,), (0,)), ((), ()))
_DNT = (((1,), (1,)), ((), ()))


def _fused_kernel(a_feat_ref, v_feat_ref, mask_a_ref, mask_v_ref,
                  modal_ref, time_ref, pad_ref,
                  Wa_ref, ba_ref, Wv_ref, bv_ref,
                  out_ref,
                  base_a_ref, base_v_ref, wpad_a_ref, wpad_v_ref,
                  Wa_bf_ref, Wv_bf_ref):
    b = pl.program_id(0)

    @pl.when(b == 0)
    def _():
        te = time_ref[...]
        ea = te + modal_ref[0:1, :]
        ev = te + modal_ref[1:2, :]
        Wae = Wa_ref[:, AD:]
        Wve = Wv_ref[:, AD:]
        base_a_ref[...] = (
            jax.lax.dot_general(ea, Wae, _DNT, preferred_element_type=jnp.float32)
            + ba_ref[...])
        base_v_ref[...] = (
            jax.lax.dot_general(ev, Wve, _DNT, preferred_element_type=jnp.float32)
            + bv_ref[...])
        wpad_a_ref[...] = jax.lax.dot_general(
            pad_ref[...], Wae, _DNT, preferred_element_type=jnp.float32)
        wpad_v_ref[...] = jax.lax.dot_general(
            pad_ref[...], Wve, _DNT, preferred_element_type=jnp.float32)
        Wa_bf_ref[...] = jnp.transpose(Wa_ref[:, :AD], (1, 0)).astype(jnp.bfloat16)
        Wv_bf_ref[...] = jnp.transpose(Wv_ref[:, :AD], (1, 0)).astype(jnp.bfloat16)

    a_feat = a_feat_ref[...].reshape(BB * T, AD).astype(jnp.bfloat16)
    v_feat = v_feat_ref[...].reshape(BB * T, AD).astype(jnp.bfloat16)
    a_mm = jax.lax.dot_general(a_feat, Wa_bf_ref[...], _DN,
                               preferred_element_type=jnp.float32)
    v_mm = jax.lax.dot_general(v_feat, Wv_bf_ref[...], _DN,
                               preferred_element_type=jnp.float32)
    for i in range(BB):
        mask_col_a = jnp.transpose(mask_a_ref[i], (1, 0))
        mask_col_v = jnp.transpose(mask_v_ref[i], (1, 0))
        out_ref[i, :T, :] = (a_mm[i * T:(i + 1) * T]
                             + base_a_ref[...]
                             + mask_col_a * wpad_a_ref[...])
        out_ref[i, T:, :] = (v_mm[i * T:(i + 1) * T]
                             + base_v_ref[...]
                             + mask_col_v * wpad_v_ref[...])


@jax.jit
def kernel(audio_feat, video_feat, mask_audio, mask_video, modal_emb,
           time_emb, pad_emb, W_audio, b_audio, W_video, b_video):
    mask_a = mask_audio.astype(jnp.float32).reshape(B, 1, T)
    mask_v = mask_video.astype(jnp.float32).reshape(B, 1, T)
    ba = b_audio.reshape(1, OD)
    bv = b_video.reshape(1, OD)

    out_tr = pl.pallas_call(
        _fused_kernel,
        grid=(B // BB,),
        in_specs=[
            pl.BlockSpec((BB, T, AD), lambda b: (b, 0, 0)),
            pl.BlockSpec((BB, T, AD), lambda b: (b, 0, 0)),
            pl.BlockSpec((BB, 1, T), lambda b: (b, 0, 0)),
            pl.BlockSpec((BB, 1, T), lambda b: (b, 0, 0)),
            pl.BlockSpec((2, ED), lambda b: (0, 0)),
            pl.BlockSpec((T, ED), lambda b: (0, 0)),
            pl.BlockSpec((1, ED), lambda b: (0, 0)),
            pl.BlockSpec((OD, AD + ED), lambda b: (0, 0)),
            pl.BlockSpec((1, OD), lambda b: (0, 0)),
            pl.BlockSpec((OD, AD + ED), lambda b: (0, 0)),
            pl.BlockSpec((1, OD), lambda b: (0, 0)),
        ],
        out_specs=pl.BlockSpec((BB, 2 * T, OD), lambda b: (b, 0, 0)),
        out_shape=jax.ShapeDtypeStruct((B, 2 * T, OD), jnp.float32),
        scratch_shapes=[
            pltpu.VMEM((T, OD), jnp.float32),
            pltpu.VMEM((T, OD), jnp.float32),
            pltpu.VMEM((1, OD), jnp.float32),
            pltpu.VMEM((1, OD), jnp.float32),
            pltpu.VMEM((AD, OD), jnp.bfloat16),
            pltpu.VMEM((AD, OD), jnp.bfloat16),
        ],
    )(audio_feat, video_feat, mask_a, mask_v, modal_emb, time_emb, pad_emb,
      W_audio, ba, W_video, bv)
    return jnp.transpose(out_tr, (0, 2, 1))


# speedup vs baseline: 4.7905x; 1.0370x over previous
import functools

import jax
import jax.numpy as jnp
from jax.experimental import pallas as pl
from jax.experimental.pallas import tpu as pltpu

B = 64
T = 200
AD = 128
ED = 128
OD = 2048

BB = 4

_DN = (((1,), (0,)), ((), ()))
_DNT = (((1,), (1,)), ((), ()))


def _fused_kernel(a_feat_ref, v_feat_ref, mask_a_ref, mask_v_ref,
                  modal_ref, time_ref, pad_ref,
                  Wa_ref, ba_ref, Wv_ref, bv_ref,
                  out_ref,
                  base_a_ref, base_v_ref, wpad_a_ref, wpad_v_ref,
                  Wa_bf_ref, Wv_bf_ref):
    b = pl.program_id(0)

    @pl.when(b == 0)
    def _():
        te = time_ref[...]
        ea = te + modal_ref[0:1, :]
        ev = te + modal_ref[1:2, :]
        Wae = Wa_ref[:, AD:]
        Wve = Wv_ref[:, AD:]
        base_a_ref[...] = (
            jax.lax.dot_general(ea, Wae, _DNT, preferred_element_type=jnp.float32)
            + ba_ref[...])
        base_v_ref[...] = (
            jax.lax.dot_general(ev, Wve, _DNT, preferred_element_type=jnp.float32)
            + bv_ref[...])
        wpad_a_ref[...] = jax.lax.dot_general(
            pad_ref[...], Wae, _DNT, preferred_element_type=jnp.float32)
        wpad_v_ref[...] = jax.lax.dot_general(
            pad_ref[...], Wve, _DNT, preferred_element_type=jnp.float32)
        Wa_bf_ref[...] = jnp.transpose(Wa_ref[:, :AD], (1, 0)).astype(jnp.bfloat16)
        Wv_bf_ref[...] = jnp.transpose(Wv_ref[:, :AD], (1, 0)).astype(jnp.bfloat16)

    a_feat = a_feat_ref[...].reshape(BB * T, AD).astype(jnp.bfloat16)
    v_feat = v_feat_ref[...].reshape(BB * T, AD).astype(jnp.bfloat16)
    a_mm = jax.lax.dot_general(a_feat, Wa_bf_ref[...], _DN,
                               preferred_element_type=jnp.float32)
    v_mm = jax.lax.dot_general(v_feat, Wv_bf_ref[...], _DN,
                               preferred_element_type=jnp.float32)
    for i in range(BB):
        mask_col_a = jnp.transpose(mask_a_ref[i], (1, 0))
        mask_col_v = jnp.transpose(mask_v_ref[i], (1, 0))
        out_ref[i, :T, :] = (a_mm[i * T:(i + 1) * T]
                             + base_a_ref[...]
                             + mask_col_a * wpad_a_ref[...])
        out_ref[i, T:, :] = (v_mm[i * T:(i + 1) * T]
                             + base_v_ref[...]
                             + mask_col_v * wpad_v_ref[...])


@jax.jit
def kernel(audio_feat, video_feat, mask_audio, mask_video, modal_emb,
           time_emb, pad_emb, W_audio, b_audio, W_video, b_video):
    mask_a = mask_audio.astype(jnp.float32).reshape(B, 1, T)
    mask_v = mask_video.astype(jnp.float32).reshape(B, 1, T)
    ba = b_audio.reshape(1, OD)
    bv = b_video.reshape(1, OD)

    out_tr = pl.pallas_call(
        _fused_kernel,
        grid=(B // BB,),
        in_specs=[
            pl.BlockSpec((BB, T, AD), lambda b: (b, 0, 0)),
            pl.BlockSpec((BB, T, AD), lambda b: (b, 0, 0)),
            pl.BlockSpec((BB, 1, T), lambda b: (b, 0, 0)),
            pl.BlockSpec((BB, 1, T), lambda b: (b, 0, 0)),
            pl.BlockSpec((2, ED), lambda b: (0, 0)),
            pl.BlockSpec((T, ED), lambda b: (0, 0)),
            pl.BlockSpec((1, ED), lambda b: (0, 0)),
            pl.BlockSpec((OD, AD + ED), lambda b: (0, 0)),
            pl.BlockSpec((1, OD), lambda b: (0, 0)),
            pl.BlockSpec((OD, AD + ED), lambda b: (0, 0)),
            pl.BlockSpec((1, OD), lambda b: (0, 0)),
        ],
        out_specs=pl.BlockSpec((BB, 2 * T, OD), lambda b: (b, 0, 0)),
        out_shape=jax.ShapeDtypeStruct((B, 2 * T, OD), jnp.float32),
        scratch_shapes=[
            pltpu.VMEM((T, OD), jnp.float32),
            pltpu.VMEM((T, OD), jnp.float32),
            pltpu.VMEM((1, OD), jnp.float32),
            pltpu.VMEM((1, OD), jnp.float32),
            pltpu.VMEM((AD, OD), jnp.bfloat16),
            pltpu.VMEM((AD, OD), jnp.bfloat16),
        ],
    )(audio_feat, video_feat, mask_a, mask_v, modal_emb, time_emb, pad_emb,
      W_audio, ba, W_video, bv)
    return jnp.transpose(out_tr, (0, 2, 1))
